# baseline (speedup 1.0000x reference)
import functools

import jax
import jax.numpy as jnp
from jax import lax
from jax.experimental import pallas as pl
from jax.experimental.pallas import tpu as pltpu
from jax.experimental.pallas import tpu_sc as plsc

NC = 2
NS = 16
NW = NC * NS
L = 16

NPAD = 102400
SLICE = NPAD // NS
E_ROWS = 50000
R = 16
NCHUNKS = E_ROWS // R
CPW = NCHUNKS // NW
EXTRA = NCHUNKS - CPW * NW

_mesh = plsc.VectorSubcoreMesh(core_axis_name="c", subcore_axis_name="s")
_sc_params = pltpu.CompilerParams(needs_layout_passes=False,
                                  use_tc_tiling_on_sc=False)


def _fill_vmem(buf, nwords, vec):
    def body(i, _):
        buf[pl.ds(i * vec.shape[0], vec.shape[0])] = vec
        return 0

    lax.fori_loop(0, nwords // vec.shape[0], body, 0)


def _worker_chunks(wid):
    c0 = wid * CPW + jnp.minimum(wid, EXTRA)
    nch = CPW + (wid < EXTRA).astype(jnp.int32)
    return c0, nch


@functools.partial(
    pl.kernel,
    out_type=jax.ShapeDtypeStruct((NW * SLICE,), jnp.float32),
    mesh=_mesh,
    scratch_types=[
        pltpu.VMEM((1, R * 128), jnp.int32),
        pltpu.VMEM((1, R * 128), jnp.int32),
        pltpu.VMEM((R * 128,), jnp.float32),
        pltpu.VMEM((SLICE,), jnp.float32),
        pltpu.VMEM_SHARED((NPAD,), jnp.float32),
        pltpu.SemaphoreType.DMA,
        pltpu.SemaphoreType.DMA,
        pltpu.SemaphoreType.DMA,
        pltpu.SemaphoreType.DMA,
    ],
    compiler_params=_sc_params,
)
def _degree_kernel(dst_hbm, out_hbm, idx_a, idx_b, ones_v, bounce_v, acc_sh,
                   sem_a, sem_b, sem_sa, sem_sb):
    c = lax.axis_index("c")
    s = lax.axis_index("s")
    wid = c * NS + s

    _fill_vmem(ones_v, R * 128, jnp.ones((L,), jnp.float32))
    _fill_vmem(bounce_v, SLICE, jnp.zeros((L,), jnp.float32))
    pltpu.sync_copy(bounce_v, acc_sh.at[pl.ds(s * SLICE, SLICE)])
    plsc.subcore_barrier()

    c0, nch = _worker_chunks(wid)
    npair = nch // 2
    odd = nch - 2 * npair

    def stage(ch, buf, sem):
        pltpu.async_copy(dst_hbm.at[pl.ds(ch, 1)], buf, sem)

    def wait_stage(ch, buf, sem):
        pltpu.make_async_copy(dst_hbm.at[pl.ds(ch, 1)], buf,
                              sem).wait()

    def streams(buf, sem_s):
        return [pltpu.async_copy(ones_v, acc_sh.at[buf.at[0]], sem_s,
                                 add=True)]

    stage(c0, idx_a, sem_a)
    stage(c0 + 1, idx_b, sem_b)

    def pair(t, _):
        ca = c0 + 2 * t
        wait_stage(ca, idx_a, sem_a)
        da = streams(idx_a, sem_sa)
        wait_stage(ca + 1, idx_b, sem_b)
        db = streams(idx_b, sem_sb)
        for d in da:
            d.wait()

        @pl.when(2 * t + 2 < nch)
        def _():
            stage(ca + 2, idx_a, sem_a)

        for d in db:
            d.wait()

        @pl.when(2 * t + 3 < nch)
        def _():
            stage(ca + 3, idx_b, sem_b)

        return 0

    lax.fori_loop(0, npair, pair, 0)

    @pl.when(odd == 1)
    def _():
        wait_stage(c0 + 2 * npair, idx_a, sem_a)
        for d in streams(idx_a, sem_sa):
            d.wait()

    plsc.subcore_barrier()

    pltpu.sync_copy(acc_sh.at[pl.ds(s * SLICE, SLICE)], bounce_v)
    pltpu.sync_copy(bounce_v, out_hbm.at[pl.ds(wid * SLICE, SLICE)])


@functools.partial(
    pl.kernel,
    out_type=jax.ShapeDtypeStruct((NW * SLICE,), jnp.float32),
    mesh=_mesh,
    scratch_types=[
        pltpu.VMEM((R, 128), jnp.int32),
        pltpu.VMEM((R, 128), jnp.int32),
        pltpu.VMEM((R, 128), jnp.int32),
        pltpu.VMEM((R, 128), jnp.int32),
        pltpu.VMEM((R, 128), jnp.float32),
        pltpu.VMEM((R, 128), jnp.float32),
        pltpu.VMEM((SLICE,), jnp.float32),
        pltpu.VMEM((NPAD,), jnp.float32),
        pltpu.VMEM_SHARED((NPAD,), jnp.float32),
        pltpu.SemaphoreType.DMA,
        pltpu.SemaphoreType.DMA,
        pltpu.SemaphoreType.DMA,
        pltpu.SemaphoreType.DMA,
    ],
    compiler_params=_sc_params,
)
def _scatter_kernel(srcr_hbm, dstr_hbm, y_hbm, out_hbm,
                    src_a, src_b, dst_a, dst_b, msg_a, msg_b,
                    bounce_v, y_v, acc_sh, sem_a, sem_b, sem_sa, sem_sb):
    c = lax.axis_index("c")
    s = lax.axis_index("s")
    wid = c * NS + s

    pltpu.sync_copy(y_hbm, y_v)
    _fill_vmem(bounce_v, SLICE, jnp.zeros((L,), jnp.float32))
    pltpu.sync_copy(bounce_v, acc_sh.at[pl.ds(s * SLICE, SLICE)])
    plsc.subcore_barrier()

    c0, nch = _worker_chunks(wid)
    npair = nch // 2
    odd = nch - 2 * npair

    def stage(ch, src_v, dst_v, sem):
        pltpu.async_copy(srcr_hbm.at[pl.ds(R * ch, R)], src_v, sem)
        pltpu.async_copy(dstr_hbm.at[pl.ds(R * ch, R)], dst_v, sem)

    def wait_stage(ch, src_v, dst_v, sem):
        pltpu.make_async_copy(srcr_hbm.at[pl.ds(R * ch, R)], src_v,
                              sem).wait()
        pltpu.make_async_copy(dstr_hbm.at[pl.ds(R * ch, R)], dst_v,
                              sem).wait()

    def gather_streams(src_v, dst_v, msg_v, sem_s):
        descs = []
        for j in range(R):
            idxs = [src_v[j, pl.ds(k * L, L)] for k in range(128 // L)]
            msgs = [plsc.load_gather(y_v, [idx]) for idx in idxs]
            for k in range(128 // L):
                msg_v[j, pl.ds(k * L, L)] = msgs[k]
            descs.append(pltpu.async_copy(msg_v.at[j],
                                          acc_sh.at[dst_v.at[j]],
                                          sem_s, add=True))
        return descs

    stage(c0, src_a, dst_a, sem_a)
    stage(c0 + 1, src_b, dst_b, sem_b)

    def pair(t, _):
        ca = c0 + 2 * t
        wait_stage(ca, src_a, dst_a, sem_a)
        da = gather_streams(src_a, dst_a, msg_a, sem_sa)
        wait_stage(ca + 1, src_b, dst_b, sem_b)
        db = gather_streams(src_b, dst_b, msg_b, sem_sb)
        for d in da:
            d.wait()

        @pl.when(2 * t + 2 < nch)
        def _():
            stage(ca + 2, src_a, dst_a, sem_a)

        for d in db:
            d.wait()

        @pl.when(2 * t + 3 < nch)
        def _():
            stage(ca + 3, src_b, dst_b, sem_b)

        return 0

    lax.fori_loop(0, npair, pair, 0)

    @pl.when(odd == 1)
    def _():
        wait_stage(c0 + 2 * npair, src_a, dst_a, sem_a)
        for d in gather_streams(src_a, dst_a, msg_a, sem_sa):
            d.wait()

    plsc.subcore_barrier()

    pltpu.sync_copy(acc_sh.at[pl.ds(s * SLICE, SLICE)], bounce_v)
    pltpu.sync_copy(bounce_v, out_hbm.at[pl.ds(wid * SLICE, SLICE)])


def _node_prep_body(x0, x1, x2, dp, w_ref, y, dis, selfterm):
    nb = NPAD // 128
    deg = dp[0:nb, :] + dp[nb:2 * nb, :] + 1.0
    d = lax.rsqrt(deg)
    xw = (x0[:, :] * w_ref[0, 0] + x1[:, :] * w_ref[1, 0]
          + x2[:, :] * w_ref[2, 0])
    dis[:, :] = d
    y[:, :] = xw * d
    selfterm[:, :] = xw / deg


def _combine_body(sp, dis, selfterm, b_ref, out):
    nb = NPAD // 128
    out[:, :] = (dis[:, :] * (sp[0:nb, :] + sp[nb:2 * nb, :])
                 + selfterm[:, :] + b_ref[0, 0])


def kernel(x, edge_index, W, b):
    n = x.shape[0]

    ei = edge_index.astype(jnp.int32)
    dst2d = ei[1].reshape(E_ROWS, 128)
    ei2 = lax.optimization_barrier(ei)
    src2d = ei2[0].reshape(E_ROWS, 128)

    xt = jnp.pad(x, ((0, NPAD - n), (0, 0))).T
    x0 = xt[0].reshape(NPAD // 128, 128)
    x1 = xt[1].reshape(NPAD // 128, 128)
    x2 = xt[2].reshape(NPAD // 128, 128)

    degparts = _degree_kernel(dst2d.reshape(NCHUNKS, R * 128))
    dp2d = degparts.reshape(NC * NPAD // 128, 128)

    grid2d = (NPAD // 128, 128)
    vspec = pl.BlockSpec(memory_space=pltpu.VMEM)
    sspec = pl.BlockSpec(memory_space=pltpu.SMEM)
    y, dis, selfterm = pl.pallas_call(
        _node_prep_body,
        out_shape=[jax.ShapeDtypeStruct(grid2d, jnp.float32)] * 3,
        in_specs=[vspec] * 4 + [sspec],
    )(x0, x1, x2, dp2d, W)

    sparts = _scatter_kernel(src2d, dst2d, y.reshape(NPAD))
    sp2d = sparts.reshape(NC * NPAD // 128, 128)

    out2d = pl.pallas_call(
        _combine_body,
        out_shape=jax.ShapeDtypeStruct(grid2d, jnp.float32),
        in_specs=[vspec] * 3 + [sspec],
    )(sp2d, dis, selfterm, b.reshape(1, 1))

    return out2d.reshape(NPAD, 1)[:n]


# speedup vs baseline: 1.0144x; 1.0144x over previous
import functools

import jax
import jax.numpy as jnp
from jax import lax
from jax.experimental import pallas as pl
from jax.experimental.pallas import tpu as pltpu
from jax.experimental.pallas import tpu_sc as plsc

NC = 2
NS = 16
NW = NC * NS
L = 16

NPAD = 102400
SLICE = NPAD // NS
E_ROWS = 50000
R = 16
NCHUNKS = E_ROWS // R
CPW = NCHUNKS // NW
EXTRA = NCHUNKS - CPW * NW

_mesh = plsc.VectorSubcoreMesh(core_axis_name="c", subcore_axis_name="s")
_sc_params = pltpu.CompilerParams(needs_layout_passes=False,
                                  use_tc_tiling_on_sc=False)


def _fill_vmem(buf, nwords, vec):
    def body(i, _):
        buf[pl.ds(i * vec.shape[0], vec.shape[0])] = vec
        return 0

    lax.fori_loop(0, nwords // vec.shape[0], body, 0)


def _worker_chunks(wid):
    c0 = wid * CPW + jnp.minimum(wid, EXTRA)
    nch = CPW + (wid < EXTRA).astype(jnp.int32)
    return c0, nch


@functools.partial(
    pl.kernel,
    out_type=jax.ShapeDtypeStruct((NW * SLICE,), jnp.float32),
    mesh=_mesh,
    scratch_types=[
        pltpu.VMEM((R, 128), jnp.int32),
        pltpu.VMEM((R, 128), jnp.int32),
        pltpu.VMEM((128,), jnp.float32),
        pltpu.VMEM((SLICE,), jnp.float32),
        pltpu.VMEM_SHARED((NPAD,), jnp.float32),
        pltpu.SemaphoreType.DMA,
        pltpu.SemaphoreType.DMA,
        pltpu.SemaphoreType.DMA,
        pltpu.SemaphoreType.DMA,
    ],
    compiler_params=_sc_params,
)
def _degree_kernel(dst_hbm, out_hbm, idx_a, idx_b, ones_v, bounce_v, acc_sh,
                   sem_a, sem_b, sem_sa, sem_sb):
    c = lax.axis_index("c")
    s = lax.axis_index("s")
    wid = c * NS + s

    _fill_vmem(ones_v, 128, jnp.ones((L,), jnp.float32))
    _fill_vmem(bounce_v, SLICE, jnp.zeros((L,), jnp.float32))
    pltpu.sync_copy(bounce_v, acc_sh.at[pl.ds(s * SLICE, SLICE)])
    plsc.subcore_barrier()

    c0, nch = _worker_chunks(wid)
    npair = nch // 2
    odd = nch - 2 * npair

    def stage(ch, buf, sem):
        pltpu.async_copy(dst_hbm.at[pl.ds(R * ch, R)], buf, sem)

    def wait_stage(ch, buf, sem):
        pltpu.make_async_copy(dst_hbm.at[pl.ds(R * ch, R)], buf,
                              sem).wait()

    def streams(buf, sem_s):
        return [pltpu.async_copy(ones_v, acc_sh.at[buf.at[j]], sem_s,
                                 add=True) for j in range(R)]

    stage(c0, idx_a, sem_a)
    stage(c0 + 1, idx_b, sem_b)

    def pair(t, _):
        ca = c0 + 2 * t
        wait_stage(ca, idx_a, sem_a)
        da = streams(idx_a, sem_sa)
        wait_stage(ca + 1, idx_b, sem_b)
        db = streams(idx_b, sem_sb)
        for d in da:
            d.wait()

        @pl.when(2 * t + 2 < nch)
        def _():
            stage(ca + 2, idx_a, sem_a)

        for d in db:
            d.wait()

        @pl.when(2 * t + 3 < nch)
        def _():
            stage(ca + 3, idx_b, sem_b)

        return 0

    lax.fori_loop(0, npair, pair, 0)

    @pl.when(odd == 1)
    def _():
        wait_stage(c0 + 2 * npair, idx_a, sem_a)
        for d in streams(idx_a, sem_sa):
            d.wait()

    plsc.subcore_barrier()

    pltpu.sync_copy(acc_sh.at[pl.ds(s * SLICE, SLICE)], bounce_v)
    pltpu.sync_copy(bounce_v, out_hbm.at[pl.ds(wid * SLICE, SLICE)])


@functools.partial(
    pl.kernel,
    out_type=jax.ShapeDtypeStruct((NW * SLICE,), jnp.float32),
    mesh=_mesh,
    scratch_types=[
        pltpu.VMEM((R, 128), jnp.int32),
        pltpu.VMEM((R, 128), jnp.int32),
        pltpu.VMEM((R, 128), jnp.int32),
        pltpu.VMEM((R, 128), jnp.int32),
        pltpu.VMEM((R, 128), jnp.float32),
        pltpu.VMEM((R, 128), jnp.float32),
        pltpu.VMEM((SLICE,), jnp.float32),
        pltpu.VMEM((NPAD,), jnp.float32),
        pltpu.VMEM_SHARED((NPAD,), jnp.float32),
        pltpu.SemaphoreType.DMA,
        pltpu.SemaphoreType.DMA,
        pltpu.SemaphoreType.DMA,
        pltpu.SemaphoreType.DMA,
        pltpu.SemaphoreType.DMA,
    ],
    compiler_params=_sc_params,
)
def _scatter_kernel(srcr_hbm, dstr_hbm, y_hbm, out_hbm,
                    src_a, src_b, dst_a, dst_b, msg_a, msg_b,
                    bounce_v, y_v, acc_sh, sem_a, sem_b, sem_sa, sem_sb,
                    sem_y):
    c = lax.axis_index("c")
    s = lax.axis_index("s")
    wid = c * NS + s

    y_copy = pltpu.async_copy(y_hbm, y_v, sem_y)
    _fill_vmem(bounce_v, SLICE, jnp.zeros((L,), jnp.float32))
    pltpu.sync_copy(bounce_v, acc_sh.at[pl.ds(s * SLICE, SLICE)])
    plsc.subcore_barrier()

    c0, nch = _worker_chunks(wid)
    npair = nch // 2
    odd = nch - 2 * npair

    def stage(ch, src_v, dst_v, sem):
        pltpu.async_copy(srcr_hbm.at[pl.ds(R * ch, R)], src_v, sem)
        pltpu.async_copy(dstr_hbm.at[pl.ds(R * ch, R)], dst_v, sem)

    def wait_stage(ch, src_v, dst_v, sem):
        pltpu.make_async_copy(srcr_hbm.at[pl.ds(R * ch, R)], src_v,
                              sem).wait()
        pltpu.make_async_copy(dstr_hbm.at[pl.ds(R * ch, R)], dst_v,
                              sem).wait()

    def gather_streams(src_v, dst_v, msg_v, sem_s):
        descs = []
        for j in range(R):
            idxs = [src_v[j, pl.ds(k * L, L)] for k in range(128 // L)]
            msgs = [plsc.load_gather(y_v, [idx]) for idx in idxs]
            for k in range(128 // L):
                msg_v[j, pl.ds(k * L, L)] = msgs[k]
            descs.append(pltpu.async_copy(msg_v.at[j],
                                          acc_sh.at[dst_v.at[j]],
                                          sem_s, add=True))
        return descs

    stage(c0, src_a, dst_a, sem_a)
    stage(c0 + 1, src_b, dst_b, sem_b)
    y_copy.wait()

    def pair(t, _):
        ca = c0 + 2 * t
        wait_stage(ca, src_a, dst_a, sem_a)
        da = gather_streams(src_a, dst_a, msg_a, sem_sa)
        wait_stage(ca + 1, src_b, dst_b, sem_b)
        db = gather_streams(src_b, dst_b, msg_b, sem_sb)
        for d in da:
            d.wait()

        @pl.when(2 * t + 2 < nch)
        def _():
            stage(ca + 2, src_a, dst_a, sem_a)

        for d in db:
            d.wait()

        @pl.when(2 * t + 3 < nch)
        def _():
            stage(ca + 3, src_b, dst_b, sem_b)

        return 0

    lax.fori_loop(0, npair, pair, 0)

    @pl.when(odd == 1)
    def _():
        wait_stage(c0 + 2 * npair, src_a, dst_a, sem_a)
        for d in gather_streams(src_a, dst_a, msg_a, sem_sa):
            d.wait()

    plsc.subcore_barrier()

    pltpu.sync_copy(acc_sh.at[pl.ds(s * SLICE, SLICE)], bounce_v)
    pltpu.sync_copy(bounce_v, out_hbm.at[pl.ds(wid * SLICE, SLICE)])


def _node_prep_body(x0, x1, x2, dp, w_ref, y, dis, selfterm):
    nb = NPAD // 128
    deg = dp[0:nb, :] + dp[nb:2 * nb, :] + 1.0
    d = lax.rsqrt(deg)
    xw = (x0[:, :] * w_ref[0, 0] + x1[:, :] * w_ref[1, 0]
          + x2[:, :] * w_ref[2, 0])
    dis[:, :] = d
    y[:, :] = xw * d
    selfterm[:, :] = xw / deg


def _combine_body(sp, dis, selfterm, b_ref, out):
    nb = NPAD // 128
    out[:, :] = (dis[:, :] * (sp[0:nb, :] + sp[nb:2 * nb, :])
                 + selfterm[:, :] + b_ref[0, 0])


def kernel(x, edge_index, W, b):
    n = x.shape[0]

    ei = edge_index.astype(jnp.int32)
    dst2d = ei[1].reshape(E_ROWS, 128)
    ei2 = lax.optimization_barrier(ei)
    src2d = ei2[0].reshape(E_ROWS, 128)

    xt = jnp.pad(x, ((0, NPAD - n), (0, 0))).T
    x0 = xt[0].reshape(NPAD // 128, 128)
    x1 = xt[1].reshape(NPAD // 128, 128)
    x2 = xt[2].reshape(NPAD // 128, 128)

    degparts = _degree_kernel(dst2d)
    dp2d = degparts.reshape(NC * NPAD // 128, 128)

    grid2d = (NPAD // 128, 128)
    vspec = pl.BlockSpec(memory_space=pltpu.VMEM)
    sspec = pl.BlockSpec(memory_space=pltpu.SMEM)
    y, dis, selfterm = pl.pallas_call(
        _node_prep_body,
        out_shape=[jax.ShapeDtypeStruct(grid2d, jnp.float32)] * 3,
        in_specs=[vspec] * 4 + [sspec],
    )(x0, x1, x2, dp2d, W)

    sparts = _scatter_kernel(src2d, dst2d, y.reshape(NPAD))
    sp2d = sparts.reshape(NC * NPAD // 128, 128)

    out2d = pl.pallas_call(
        _combine_body,
        out_shape=jax.ShapeDtypeStruct(grid2d, jnp.float32),
        in_specs=[vspec] * 3 + [sspec],
    )(sp2d, dis, selfterm, b.reshape(1, 1))

    return out2d.reshape(NPAD, 1)[:n]
